# baseline (speedup 1.0000x reference)
import functools

import jax
import jax.numpy as jnp
from jax import lax
from jax.experimental import pallas as pl
from jax.experimental.pallas import tpu as pltpu
from jax.experimental.pallas import tpu_sc as plsc

V = 1_000_000
D = 64
B = 4096
L = 50

NC = 2
NS = 16
LANES = 16
NW = NC * NS
QPW = B // NW
NQG = QPW // LANES
ND = D // LANES
RW = 2 * D
NT = 2
CHUNKS = [(tb, min(NT, L - tb)) for tb in range(0, L, NT)]

_GDN = lax.GatherDimensionNumbers(
    offset_dims=(), collapsed_slice_dims=(0,), start_index_map=(0,))


def _bcast_lane(vec, lane):
    idx = jnp.full((LANES, 1), lane, jnp.int32)
    return lax.gather(vec, idx, dimension_numbers=_GDN, slice_sizes=(1,),
                      mode=lax.GatherScatterMode.PROMISE_IN_BOUNDS)


def _perm(vec, idx):
    return lax.gather(vec, idx[:, None], dimension_numbers=_GDN,
                      slice_sizes=(1,),
                      mode=lax.GatherScatterMode.PROMISE_IN_BOUNDS)


def _all_reduce(vec, op, iota):
    for k in (8, 4, 2, 1):
        vec = op(vec, _perm(vec, iota ^ k))
    return vec


def _softmax_body(query_hbm, weights_hbm, p_hbm, s_hbm,
                  idx_v, w_v, p_v, s_v, sem_w, sem_o):
    wid = lax.axis_index("s") * NC + lax.axis_index("c")
    q0 = wid * QPW

    icopies = [pltpu.async_copy(
        query_hbm.at[pl.ds(t * B + q0, QPW)], idx_v.at[t], sem_o)
        for t in range(L)]
    for c in icopies:
        c.wait()
    wcopies = [pltpu.async_copy(
        weights_hbm.at[idx_v.at[t]], w_v.at[t], sem_w) for t in range(L)]
    for c in wcopies:
        c.wait()

    for qg in range(NQG):
        qoff = qg * LANES

        def wrow(t):
            return w_v[t, pl.ds(qoff, LANES)]

        m = wrow(0)
        for t in range(1, L):
            m = jnp.maximum(m, wrow(t))
        s = jnp.zeros((LANES,), jnp.float32)
        for t in range(L):
            e = jnp.exp(wrow(t) - m)
            p_v[t, pl.ds(qoff, LANES)] = e
            s = s + e
        s_v[pl.ds(qoff, LANES)] = s

    ocopies = [pltpu.async_copy(
        p_v.at[t], p_hbm.at[pl.ds(t * B + q0, QPW)], sem_o)
        for t in range(L)]
    ocopies.append(pltpu.async_copy(s_v, s_hbm.at[pl.ds(q0, QPW)], sem_o))
    for c in ocopies:
        c.wait()


def _pool_body(query_hbm, table_hbm, p_hbm, s_hbm, bias_hbm, out_hbm,
               idx_v, p_v, s_v, rows_a, rows_b, out_v,
               bias_v, sem_m, sem_a, sem_b):
    wid = lax.axis_index("s") * NC + lax.axis_index("c")
    q0 = wid * QPW
    iota = lax.iota(jnp.int32, LANES)

    mcopies = [pltpu.async_copy(
        query_hbm.at[pl.ds(t * B + q0, QPW)], idx_v.at[t], sem_m)
        for t in range(L)]
    mcopies.append(pltpu.async_copy(bias_hbm, bias_v, sem_m))
    for c in mcopies:
        c.wait()

    rows_bufs = [rows_a, rows_b]
    sems = [sem_a, sem_b]

    def fire_chunk(ci):
        tb, nt = CHUNKS[ci]
        buf, sem = rows_bufs[ci % 2], sems[ci % 2]
        return [pltpu.async_copy(
            table_hbm.at[idx_v.at[tb + j]],
            buf.at[pl.ds(j * QPW, QPW)], sem) for j in range(nt)]

    inflight = fire_chunk(0)

    mcopies = [pltpu.async_copy(
        p_hbm.at[pl.ds(t * B + q0, QPW)], p_v.at[t], sem_m)
        for t in range(L)]
    mcopies.append(pltpu.async_copy(s_hbm.at[pl.ds(q0, QPW)], s_v, sem_m))
    for c in mcopies:
        c.wait()
    bias_regs = [bias_v[pl.ds(d * LANES, LANES)] for d in range(ND)]

    for ci, (tb, nt) in enumerate(CHUNKS):
        for c in inflight:
            c.wait()
        nxt = fire_chunk(ci + 1) if ci + 1 < len(CHUNKS) else []
        buf = rows_bufs[ci % 2]
        first = ci == 0

        def acc_q(q, _, tb=tb, nt=nt, buf=buf, first=first):
            qg16 = (q // LANES) * LANES
            lane = q % LANES
            if first:
                accs = [jnp.zeros((LANES,), jnp.float32) for _ in range(ND)]
            else:
                accs = [out_v[q, pl.ds(d * LANES, LANES)] for d in range(ND)]
            for j in range(nt):
                pb = _bcast_lane(p_v[tb + j, pl.ds(qg16, LANES)], lane)
                r = j * QPW + q
                accs = [accs[d] + pb * buf[r, pl.ds(d * LANES, LANES)]
                        for d in range(ND)]
            for d in range(ND):
                out_v[q, pl.ds(d * LANES, LANES)] = accs[d]
            return 0

        lax.fori_loop(0, QPW, acc_q, 0)
        inflight = nxt

    def fin_q(q, _):
        qg16 = (q // LANES) * LANES
        lane = q % LANES
        accs = [out_v[q, pl.ds(d * LANES, LANES)] for d in range(ND)]
        a0, a1, a2, a3 = accs
        s2v = jnp.maximum(
            _all_reduce(a0 * a0 + a1 * a1 + a2 * a2 + a3 * a3,
                        jnp.add, iota),
            jnp.float32(1e-30))
        bits = lax.bitcast_convert_type(s2v, jnp.int32)
        y = lax.bitcast_convert_type(
            jnp.int32(0x5F3759DF) - lax.shift_right_logical(bits, 1),
            jnp.float32)
        y = y * (1.5 - 0.5 * s2v * y * y)
        y = y * (1.5 - 0.5 * s2v * y * y)
        y = y * (1.5 - 0.5 * s2v * y * y)
        sg = _bcast_lane(s_v[pl.ds(qg16, LANES)], lane)
        invn = 1.0 / (s2v * y + 1e-4 * sg)
        for d in range(ND):
            out_v[q, pl.ds(d * LANES, LANES)] = (
                accs[d] * invn + bias_regs[d])
        return 0

    lax.fori_loop(0, QPW, fin_q, 0)
    pltpu.sync_copy(out_v, out_hbm.at[pl.ds(q0, QPW)])


@functools.partial(jax.jit)
def _encode(query_flat, table2, weights_flat, bias):
    mesh = plsc.VectorSubcoreMesh(core_axis_name="c", subcore_axis_name="s")
    softmax_run = functools.partial(
        pl.kernel,
        out_type=(jax.ShapeDtypeStruct((B * L,), jnp.float32),
                  jax.ShapeDtypeStruct((B,), jnp.float32)),
        mesh=mesh,
        compiler_params=pltpu.CompilerParams(use_tc_tiling_on_sc=False),
        scratch_types=[
            pltpu.VMEM((L, QPW), jnp.int32),
            pltpu.VMEM((L, QPW), jnp.float32),
            pltpu.VMEM((L, QPW), jnp.float32),
            pltpu.VMEM((QPW,), jnp.float32),
            pltpu.SemaphoreType.DMA,
            pltpu.SemaphoreType.DMA,
        ],
    )(_softmax_body)
    p_flat, s = softmax_run(query_flat, weights_flat)

    pool_run = functools.partial(
        pl.kernel,
        out_type=jax.ShapeDtypeStruct((B, D), jnp.float32),
        mesh=mesh,
        compiler_params=pltpu.CompilerParams(use_tc_tiling_on_sc=True),
        scratch_types=[
            pltpu.VMEM((L, QPW), jnp.int32),
            pltpu.VMEM((L, QPW), jnp.float32),
            pltpu.VMEM((QPW,), jnp.float32),
            pltpu.VMEM((NT * QPW, RW), jnp.float32),
            pltpu.VMEM((NT * QPW, RW), jnp.float32),
            pltpu.VMEM((QPW, D), jnp.float32),
            pltpu.VMEM((D,), jnp.float32),
            pltpu.SemaphoreType.DMA,
            pltpu.SemaphoreType.DMA,
            pltpu.SemaphoreType.DMA,
        ],
    )(_pool_body)
    return pool_run(query_flat, table2, p_flat, s, bias)


def kernel(query, table, weights, bias):
    query_flat = query.T.astype(jnp.int32).reshape(-1)
    nsec = 4
    rows = V // nsec
    table2 = jnp.concatenate(
        [jnp.pad(lax.slice(table, (i * rows, 0), ((i + 1) * rows, D)),
                 ((0, 0), (0, RW - D)))
         for i in range(nsec)], axis=0)
    return _encode(query_flat, table2, weights.reshape(-1),
                   bias.astype(jnp.float32))


# speedup vs baseline: 2.2526x; 2.2526x over previous
import functools

import jax
import jax.numpy as jnp
from jax import lax
from jax.experimental import pallas as pl
from jax.experimental.pallas import tpu as pltpu
from jax.experimental.pallas import tpu_sc as plsc

V = 1_000_000
D = 64
B = 4096
L = 50

NC = 2
NS = 16
LANES = 16
NW = NC * NS
QPW = B // NW
NQG = QPW // LANES
ND = D // LANES
RW = 2 * D
NT = 2
CHUNKS = [(tb, min(NT, L - tb)) for tb in range(0, L, NT)]

_GDN = lax.GatherDimensionNumbers(
    offset_dims=(), collapsed_slice_dims=(0,), start_index_map=(0,))


def _bcast_lane(vec, lane):
    idx = jnp.full((LANES, 1), lane, jnp.int32)
    return lax.gather(vec, idx, dimension_numbers=_GDN, slice_sizes=(1,),
                      mode=lax.GatherScatterMode.PROMISE_IN_BOUNDS)


def _perm(vec, idx):
    return lax.gather(vec, idx[:, None], dimension_numbers=_GDN,
                      slice_sizes=(1,),
                      mode=lax.GatherScatterMode.PROMISE_IN_BOUNDS)


def _all_reduce(vec, op, iota):
    for k in (8, 4, 2, 1):
        vec = op(vec, _perm(vec, iota ^ k))
    return vec


def _softmax_body(query_hbm, weights_hbm, p_hbm, s_hbm,
                  idx_v, w_v, p_v, s_v, sem_w, sem_o):
    wid = lax.axis_index("s") * NC + lax.axis_index("c")
    q0 = wid * QPW

    icopies = [pltpu.async_copy(
        query_hbm.at[pl.ds(t * B + q0, QPW)], idx_v.at[t], sem_o)
        for t in range(L)]
    for c in icopies:
        c.wait()
    wcopies = [pltpu.async_copy(
        weights_hbm.at[idx_v.at[t]], w_v.at[t], sem_w) for t in range(L)]
    for c in wcopies:
        c.wait()

    for qg in range(NQG):
        qoff = qg * LANES

        def wrow(t):
            return w_v[t, pl.ds(qoff, LANES)]

        m = wrow(0)
        for t in range(1, L):
            m = jnp.maximum(m, wrow(t))
        s = jnp.zeros((LANES,), jnp.float32)
        for t in range(L):
            e = jnp.exp(wrow(t) - m)
            p_v[t, pl.ds(qoff, LANES)] = e
            s = s + e
        s_v[pl.ds(qoff, LANES)] = s

    ocopies = [pltpu.async_copy(
        p_v.at[t], p_hbm.at[pl.ds(t * B + q0, QPW)], sem_o)
        for t in range(L)]
    ocopies.append(pltpu.async_copy(s_v, s_hbm.at[pl.ds(q0, QPW)], sem_o))
    for c in ocopies:
        c.wait()


def _pool_body(query_hbm, table_hbm, p_hbm, s_hbm, bias_hbm, out_hbm,
               idx_v, p_v, s_v, rows_a, rows_b, out_v,
               bias_v, sem_m, sem_a, sem_b):
    wid = lax.axis_index("s") * NC + lax.axis_index("c")
    q0 = wid * QPW
    iota = lax.iota(jnp.int32, LANES)

    mcopies = [pltpu.async_copy(
        query_hbm.at[pl.ds(t * B + q0, QPW)], idx_v.at[t], sem_m)
        for t in range(L)]
    mcopies.append(pltpu.async_copy(bias_hbm, bias_v, sem_m))
    for c in mcopies:
        c.wait()

    rows_bufs = [rows_a, rows_b]
    sems = [sem_a, sem_b]

    def fire_chunk(ci):
        tb, nt = CHUNKS[ci]
        buf, sem = rows_bufs[ci % 2], sems[ci % 2]
        return [pltpu.async_copy(
            table_hbm.at[idx_v.at[tb + j]],
            buf.at[pl.ds(j * QPW, QPW)], sem) for j in range(nt)]

    inflight = fire_chunk(0)

    mcopies = [pltpu.async_copy(
        p_hbm.at[pl.ds(t * B + q0, QPW)], p_v.at[t], sem_m)
        for t in range(L)]
    mcopies.append(pltpu.async_copy(s_hbm.at[pl.ds(q0, QPW)], s_v, sem_m))
    for c in mcopies:
        c.wait()
    bias_regs = [bias_v[pl.ds(d * LANES, LANES)] for d in range(ND)]

    for ci, (tb, nt) in enumerate(CHUNKS):
        for c in inflight:
            c.wait()
        nxt = fire_chunk(ci + 1) if ci + 1 < len(CHUNKS) else []
        buf = rows_bufs[ci % 2]
        first = ci == 0

        def acc_q(q, _, tb=tb, nt=nt, buf=buf, first=first):
            qg16 = (q // LANES) * LANES
            lane = q % LANES
            if first:
                accs = [jnp.zeros((LANES,), jnp.float32) for _ in range(ND)]
            else:
                accs = [out_v[q, pl.ds(d * LANES, LANES)] for d in range(ND)]
            for j in range(nt):
                pb = _bcast_lane(p_v[tb + j, pl.ds(qg16, LANES)], lane)
                r = j * QPW + q
                accs = [accs[d] + pb * buf[r, pl.ds(d * LANES, LANES)]
                        for d in range(ND)]
            for d in range(ND):
                out_v[q, pl.ds(d * LANES, LANES)] = accs[d]
            return 0

        lax.fori_loop(0, QPW, acc_q, 0)
        inflight = nxt

    def fin_q(q, _):
        qg16 = (q // LANES) * LANES
        lane = q % LANES
        accs = [out_v[q, pl.ds(d * LANES, LANES)] for d in range(ND)]
        a0, a1, a2, a3 = accs
        s2v = jnp.maximum(
            _all_reduce(a0 * a0 + a1 * a1 + a2 * a2 + a3 * a3,
                        jnp.add, iota),
            jnp.float32(1e-30))
        bits = lax.bitcast_convert_type(s2v, jnp.int32)
        y = lax.bitcast_convert_type(
            jnp.int32(0x5F3759DF) - lax.shift_right_logical(bits, 1),
            jnp.float32)
        y = y * (1.5 - 0.5 * s2v * y * y)
        y = y * (1.5 - 0.5 * s2v * y * y)
        y = y * (1.5 - 0.5 * s2v * y * y)
        sg = _bcast_lane(s_v[pl.ds(qg16, LANES)], lane)
        invn = 1.0 / (s2v * y + 1e-4 * sg)
        for d in range(ND):
            out_v[q, pl.ds(d * LANES, LANES)] = (
                accs[d] * invn + bias_regs[d])
        return 0

    lax.fori_loop(0, QPW, fin_q, 0)
    pltpu.sync_copy(out_v, out_hbm.at[pl.ds(q0, QPW)])


@functools.partial(jax.jit)
def _encode(query_flat, table2, weights_flat, bias):
    mesh = plsc.VectorSubcoreMesh(core_axis_name="c", subcore_axis_name="s")
    softmax_run = functools.partial(
        pl.kernel,
        out_type=(jax.ShapeDtypeStruct((B * L,), jnp.float32),
                  jax.ShapeDtypeStruct((B,), jnp.float32)),
        mesh=mesh,
        compiler_params=pltpu.CompilerParams(use_tc_tiling_on_sc=False),
        scratch_types=[
            pltpu.VMEM((L, QPW), jnp.int32),
            pltpu.VMEM((L, QPW), jnp.float32),
            pltpu.VMEM((L, QPW), jnp.float32),
            pltpu.VMEM((QPW,), jnp.float32),
            pltpu.SemaphoreType.DMA,
            pltpu.SemaphoreType.DMA,
        ],
    )(_softmax_body)
    p_flat, s = softmax_run(query_flat, weights_flat)

    pool_run = functools.partial(
        pl.kernel,
        out_type=jax.ShapeDtypeStruct((B, D), jnp.float32),
        mesh=mesh,
        compiler_params=pltpu.CompilerParams(use_tc_tiling_on_sc=True),
        scratch_types=[
            pltpu.VMEM((L, QPW), jnp.int32),
            pltpu.VMEM((L, QPW), jnp.float32),
            pltpu.VMEM((QPW,), jnp.float32),
            pltpu.VMEM((NT * QPW, RW), jnp.float32),
            pltpu.VMEM((NT * QPW, RW), jnp.float32),
            pltpu.VMEM((QPW, D), jnp.float32),
            pltpu.VMEM((D,), jnp.float32),
            pltpu.SemaphoreType.DMA,
            pltpu.SemaphoreType.DMA,
            pltpu.SemaphoreType.DMA,
        ],
    )(_pool_body)
    return pool_run(query_flat, table2, p_flat, s, bias)


def kernel(query, table, weights, bias):
    query_flat = query.T.astype(jnp.int32).reshape(-1)
    table2 = jnp.pad(table, ((0, 0), (0, RW - D)))
    return _encode(query_flat, table2, weights.reshape(-1),
                   bias.astype(jnp.float32))
